# baseline (speedup 1.0000x reference)
import functools

import numpy as np
import jax
import jax.numpy as jnp
from jax import lax
from jax.experimental import pallas as pl
from jax.experimental.pallas import tpu as pltpu
from jax.experimental.pallas import tpu_sc as plsc

_NC, _NS = 2, 16
_NW = _NC * _NS
_LANES = 16


def _pos_weights(J, d):
    j = np.arange(J, dtype=np.float32)[:, None] + 1.0
    k = np.arange(d, dtype=np.float32)[None, :] + 1.0
    return np.asarray(1.0 - j / J - (k / d) * (1.0 - 2.0 * j / J), np.float32)


def _pos_weight_factors(J, d):
    j = np.arange(J, dtype=np.float32) + 1.0
    k = np.arange(d, dtype=np.float32) + 1.0
    aw = 1.0 - j / J
    bw = 2.0 * j / J - 1.0
    ce = k / d
    return (np.asarray(aw, np.float32), np.asarray(bw, np.float32),
            np.asarray(ce, np.float32))


def _to_pairs(t):
    Vv, E = t.shape
    H = Vv // 2
    BT = 2000
    grid = H // BT

    def body(a_ref, b_ref, o_ref):
        o_ref[...] = jnp.concatenate([a_ref[...], b_ref[...]], axis=1)

    return pl.pallas_call(
        body,
        grid=(grid,),
        in_specs=[
            pl.BlockSpec((BT, E), lambda i: (i, 0)),
            pl.BlockSpec((BT, E), lambda i: (i + H // BT, 0)),
        ],
        out_specs=pl.BlockSpec((BT, 2 * E), lambda i: (i, 0)),
        out_shape=jax.ShapeDtypeStruct((H, 2 * E), jnp.float32),
    )(t, t)


def _sc_bags(q_idx, ctx_idx, A0, C0, C1, C2, qaw, qbw, saw, sbw, ce):
    NQB, S1 = q_idx.shape
    NSB, S2 = ctx_idx.shape
    E = A0.shape[1]
    E2 = 2 * E
    QB = NQB // _NW
    SB = NSB // _NW
    EC = E // _LANES
    QBATCH = QB
    SBATCH = 5

    S1P, S2P = S1 + _LANES, S2 + _LANES
    HV = A0.shape[0] // 2
    qhi = (q_idx % HV).reshape(_NW, QB * S1)
    qpar = jnp.pad((q_idx >= HV).astype(jnp.float32).reshape(_NW, QB, S1),
                   ((0, 0), (0, 0), (0, _LANES)))
    chi = (ctx_idx % HV).reshape(_NW, SB * S2)
    cpar = jnp.pad((ctx_idx >= HV).astype(jnp.float32).reshape(_NW, SB, S2),
                   ((0, 0), (0, 0), (0, _LANES)))
    qaw, qbw, saw, sbw = (jnp.pad(x, (0, _LANES))
                          for x in (qaw, qbw, saw, sbw))
    tabs = [_to_pairs(t) for t in (A0, C0, C1, C2)]

    mesh = plsc.VectorSubcoreMesh(core_axis_name="c", subcore_axis_name="s")

    def body(qhi_hbm, qpar_hbm, chi_hbm, cpar_hbm, a0, c0, c1, c2,
             qaw_hbm, qbw_hbm, saw_hbm, sbw_hbm, ce_hbm,
             st_out, bA0, bC0, bC1, bC2,
             qhi_v, qpar_v, chi_v, cpar_v, qaw_v, qbw_v, saw_v, sbw_v, ce_v,
             rows0, rows1, qacc_v, sacc_v, sem0, sem1):
        wid = lax.axis_index("s") * _NC + lax.axis_index("c")
        pltpu.sync_copy(qhi_hbm.at[wid], qhi_v)
        pltpu.sync_copy(qpar_hbm.at[wid], qpar_v)
        pltpu.sync_copy(chi_hbm.at[wid], chi_v)
        pltpu.sync_copy(cpar_hbm.at[wid], cpar_v)
        pltpu.sync_copy(qaw_hbm, qaw_v)
        pltpu.sync_copy(qbw_hbm, qbw_v)
        pltpu.sync_copy(saw_hbm, saw_v)
        pltpu.sync_copy(sbw_hbm, sbw_v)
        pltpu.sync_copy(ce_hbm, ce_v)

        def reduce_bag(rows_v, row0, par_ref, bag, aw_v, bw_v, nrows,
                       acc_ref):
            def sbody(s, accs):
                p = par_ref[bag, pl.ds(s, _LANES)][0]
                a = aw_v[pl.ds(s, _LANES)][0]
                bwt = bw_v[pl.ds(s, _LANES)][0]
                acc1 = list(accs[:EC])
                acc2 = list(accs[EC:])
                for c in range(EC):
                    h0 = rows_v[row0 + s, pl.ds(c * _LANES, _LANES)]
                    h1 = rows_v[row0 + s, pl.ds(E + c * _LANES, _LANES)]
                    h = h0 + p * (h1 - h0)
                    acc1[c] = acc1[c] + a * h
                    acc2[c] = acc2[c] + bwt * h
                return tuple(acc1) + tuple(acc2)
            accs = lax.fori_loop(
                0, nrows, sbody,
                tuple(jnp.zeros((_LANES,), jnp.float32)
                      for _ in range(2 * EC)))
            for c in range(EC):
                cv = ce_v[pl.ds(c * _LANES, _LANES)]
                acc_ref[bag, pl.ds(c * _LANES, _LANES)] = (
                    accs[c] + cv * accs[EC + c])

        def do_table(table, hi_v, par_ref, nbags, batch, nrows, aw_v, bw_v,
                     acc_ref, out_hbm):
            R = batch * nrows
            nbatches = nbags // batch

            def issue(j, buf, sem):
                pltpu.async_copy(table.at[hi_v.at[pl.ds(j * R, R)]],
                                 buf.at[pl.ds(0, R)], sem)

            def drain(buf, sem):
                pltpu.make_async_copy(table.at[pl.ds(0, R)],
                                      buf.at[pl.ds(0, R)], sem).wait()

            def compute(buf, kb):
                for jb in range(batch):
                    reduce_bag(buf, jb * nrows, par_ref, kb * batch + jb,
                               aw_v, bw_v, nrows, acc_ref)

            issue(0, rows0, sem0)
            if nbatches == 1:
                drain(rows0, sem0)
                compute(rows0, 0)
            else:
                assert nbatches % 2 == 0

                def pair(k, carry):
                    i0 = 2 * k
                    i1 = i0 + 1
                    issue(i1, rows1, sem1)
                    drain(rows0, sem0)
                    compute(rows0, i0)
                    issue(jnp.minimum(i1 + 1, nbatches - 1), rows0, sem0)
                    drain(rows1, sem1)
                    compute(rows1, i1)
                    return carry

                lax.fori_loop(0, nbatches // 2, pair, 0)
                drain(rows0, sem0)
            pltpu.sync_copy(acc_ref, out_hbm.at[wid])

        do_table(a0, qhi_v, qpar_v, QB, QBATCH, S1, qaw_v, qbw_v,
                 qacc_v, st_out)
        for table, out_hbm in ((a0, bA0), (c0, bC0), (c1, bC1), (c2, bC2)):
            do_table(table, chi_v, cpar_v, SB, SBATCH, S2, saw_v, sbw_v,
                     sacc_v, out_hbm)

    ROWS = SBATCH * S2
    f = pl.kernel(
        body,
        out_type=[jax.ShapeDtypeStruct((_NW, QB, E), jnp.float32)]
        + [jax.ShapeDtypeStruct((_NW, SB, E), jnp.float32)] * 4,
        mesh=mesh,
        scratch_types=[
            pltpu.VMEM((QB * S1,), jnp.int32),
            pltpu.VMEM((QB, S1P), jnp.float32),
            pltpu.VMEM((SB * S2,), jnp.int32),
            pltpu.VMEM((SB, S2P), jnp.float32),
            pltpu.VMEM((S1P,), jnp.float32),
            pltpu.VMEM((S1P,), jnp.float32),
            pltpu.VMEM((S2P,), jnp.float32),
            pltpu.VMEM((S2P,), jnp.float32),
            pltpu.VMEM((E,), jnp.float32),
            pltpu.VMEM((ROWS, E2), jnp.float32),
            pltpu.VMEM((ROWS, E2), jnp.float32),
            pltpu.VMEM((QB, E), jnp.float32),
            pltpu.VMEM((SB, E), jnp.float32),
            pltpu.SemaphoreType.DMA,
            pltpu.SemaphoreType.DMA,
        ],
    )
    outs = f(qhi, qpar, chi, cpar, *tabs, qaw, qbw, saw, sbw, ce)
    return (outs[0].reshape(NQB, E),) + tuple(
        o.reshape(NSB, E) for o in outs[1:])


def _hops(state0, bA0, bC0, bC1, bC2, TA, TC_, nq):
    N, E = state0.shape
    BM = bA0.shape[0]
    M = TA.shape[0]

    def body(st_ref, a_ref, c0_ref, c1_ref, c2_ref, ta_ref, tc_ref, out_ref):
        st = st_ref[...]
        ta = jnp.tile(ta_ref[...], (BM // M, 1))
        tc = jnp.tile(tc_ref[...], (BM // M, 1))
        r = lax.broadcasted_iota(jnp.int32, (N, BM), 0) // nq
        c = lax.broadcasted_iota(jnp.int32, (N, BM), 1) // M
        mask = (r == c).astype(jnp.float32)
        bags = [a_ref[...], c0_ref[...], c1_ref[...], c2_ref[...]]
        for i in range(3):
            mem = bags[i] + ta
            outp = bags[i + 1] + tc
            full = lax.dot_general(st, mem, (((1,), (1,)), ((), ())),
                                   preferred_element_type=jnp.float32)
            probs = full * mask
            resp = lax.dot_general(probs, outp, (((1,), (0,)), ((), ())),
                                   preferred_element_type=jnp.float32)
            st = st + resp
        out_ref[...] = st

    return pl.pallas_call(
        body,
        out_shape=jax.ShapeDtypeStruct((N, E), jnp.float32),
    )(state0, bA0, bC0, bC1, bC2, TA, TC_)


def _project(state, C2):
    N, E = state.shape
    Vv = C2.shape[0]
    NT = 2048
    grid = pl.cdiv(Vv, NT)

    def body(st_ref, c2_ref, out_ref):
        out_ref[...] = lax.dot_general(
            st_ref[...], c2_ref[...], (((1,), (1,)), ((), ())),
            preferred_element_type=jnp.float32)

    return pl.pallas_call(
        body,
        grid=(grid,),
        in_specs=[
            pl.BlockSpec((N, E), lambda i: (0, 0)),
            pl.BlockSpec((NT, E), lambda i: (i, 0)),
        ],
        out_specs=pl.BlockSpec((N, NT), lambda i: (0, i)),
        out_shape=jax.ShapeDtypeStruct((N, Vv), jnp.float32),
    )(state, C2)


def kernel(qa_ques, ctx_ques_ans, A0, C0, C1, C2, TA, TC):
    b, nq, s = qa_ques.shape
    m, s2 = ctx_ques_ans.shape[1], ctx_ques_ans.shape[2]

    q_idx = qa_ques.reshape(b * nq, s).astype(jnp.int32)
    ctx_idx = ctx_ques_ans.reshape(b * m, s2).astype(jnp.int32)
    e = A0.shape[1]
    qaw, qbw, ce = (jnp.asarray(x) for x in _pos_weight_factors(s, e))
    saw, sbw, _ = (jnp.asarray(x) for x in _pos_weight_factors(s2, e))

    state0, bA0, bC0, bC1, bC2 = _sc_bags(
        q_idx, ctx_idx, A0, C0, C1, C2, qaw, qbw, saw, sbw, ce)
    state = _hops(state0, bA0, bC0, bC1, bC2, TA, TC, nq)
    return _project(state, C2)


# speedup vs baseline: 1.4192x; 1.4192x over previous
import functools

import numpy as np
import jax
import jax.numpy as jnp
from jax import lax
from jax.experimental import pallas as pl
from jax.experimental.pallas import tpu as pltpu
from jax.experimental.pallas import tpu_sc as plsc

_NC, _NS = 2, 16
_NW = _NC * _NS
_LANES = 16


def _pos_weights(J, d):
    j = np.arange(J, dtype=np.float32)[:, None] + 1.0
    k = np.arange(d, dtype=np.float32)[None, :] + 1.0
    return np.asarray(1.0 - j / J - (k / d) * (1.0 - 2.0 * j / J), np.float32)


def _pos_weight_factors(J, d):
    j = np.arange(J, dtype=np.float32) + 1.0
    k = np.arange(d, dtype=np.float32) + 1.0
    aw = 1.0 - j / J
    bw = 2.0 * j / J - 1.0
    ce = k / d
    return (np.asarray(aw, np.float32), np.asarray(bw, np.float32),
            np.asarray(ce, np.float32))


_PBT = 2048


def _to_pairs(t):
    Vv, E = t.shape
    BT = _PBT
    grid = -(-Vv // (2 * BT))
    HP = grid * BT
    last = -(-Vv // BT) - 1

    def body(a_ref, b_ref, o_ref):
        o_ref[...] = jnp.concatenate([a_ref[...].T, b_ref[...].T], axis=1)

    return pl.pallas_call(
        body,
        grid=(grid,),
        in_specs=[
            pl.BlockSpec((E, BT), lambda i: (0, jnp.minimum(2 * i, last))),
            pl.BlockSpec((E, BT),
                         lambda i: (0, jnp.minimum(2 * i + 1, last))),
        ],
        out_specs=pl.BlockSpec((BT, 2 * E), lambda i: (i, 0)),
        out_shape=jax.ShapeDtypeStruct((HP, 2 * E), jnp.float32),
    )(t.T, t.T)


def _sc_bags(q_idx, ctx_idx, A0, C0, C1, C2, qaw, qbw, saw, sbw, ce):
    NQB, S1 = q_idx.shape
    NSB, S2 = ctx_idx.shape
    E = A0.shape[1]
    E2 = 2 * E
    QB = NQB // _NW
    SB = NSB // _NW
    EC = E // _LANES
    QBATCH = QB
    SBATCH = 5

    S1P, S2P = S1 + _LANES, S2 + _LANES
    BT = _PBT

    def split_idx(i):
        hi = (i // (2 * BT)) * BT + (i % BT)
        par = ((i // BT) & 1).astype(jnp.float32)
        return hi, par

    qhi_f, qpar_f = split_idx(q_idx)
    chi_f, cpar_f = split_idx(ctx_idx)
    qhi = qhi_f.reshape(_NW, QB * S1)
    qpar = jnp.pad(qpar_f.reshape(_NW, QB, S1),
                   ((0, 0), (0, 0), (0, _LANES)))
    chi = chi_f.reshape(_NW, SB * S2)
    cpar = jnp.pad(cpar_f.reshape(_NW, SB, S2),
                   ((0, 0), (0, 0), (0, _LANES)))
    qaw, qbw, saw, sbw = (jnp.pad(x, (0, _LANES))
                          for x in (qaw, qbw, saw, sbw))
    tabs = [_to_pairs(t) for t in (A0, C0, C1, C2)]

    mesh = plsc.VectorSubcoreMesh(core_axis_name="c", subcore_axis_name="s")

    def body(qhi_hbm, qpar_hbm, chi_hbm, cpar_hbm, a0, c0, c1, c2,
             qaw_hbm, qbw_hbm, saw_hbm, sbw_hbm, ce_hbm,
             st_out, bA0, bC0, bC1, bC2,
             qhi_v, qpar_v, chi_v, cpar_v, qaw_v, qbw_v, saw_v, sbw_v, ce_v,
             rows0, rows1, qacc_v, sacc_v, sem0, sem1):
        wid = lax.axis_index("s") * _NC + lax.axis_index("c")
        pltpu.sync_copy(qhi_hbm.at[wid], qhi_v)
        pltpu.sync_copy(qpar_hbm.at[wid], qpar_v)
        pltpu.sync_copy(chi_hbm.at[wid], chi_v)
        pltpu.sync_copy(cpar_hbm.at[wid], cpar_v)
        pltpu.sync_copy(qaw_hbm, qaw_v)
        pltpu.sync_copy(qbw_hbm, qbw_v)
        pltpu.sync_copy(saw_hbm, saw_v)
        pltpu.sync_copy(sbw_hbm, sbw_v)
        pltpu.sync_copy(ce_hbm, ce_v)

        def reduce_bag(rows_v, row0, par_ref, bag, aw_v, bw_v, nrows,
                       acc_ref):
            def sbody(s, accs):
                p = par_ref[bag, pl.ds(s, _LANES)][0]
                a = aw_v[pl.ds(s, _LANES)][0]
                bwt = bw_v[pl.ds(s, _LANES)][0]
                acc1 = list(accs[:EC])
                acc2 = list(accs[EC:])
                for c in range(EC):
                    h0 = rows_v[row0 + s, pl.ds(c * _LANES, _LANES)]
                    h1 = rows_v[row0 + s, pl.ds(E + c * _LANES, _LANES)]
                    h = h0 + p * (h1 - h0)
                    acc1[c] = acc1[c] + a * h
                    acc2[c] = acc2[c] + bwt * h
                return tuple(acc1) + tuple(acc2)
            accs = lax.fori_loop(
                0, nrows, sbody,
                tuple(jnp.zeros((_LANES,), jnp.float32)
                      for _ in range(2 * EC)))
            for c in range(EC):
                cv = ce_v[pl.ds(c * _LANES, _LANES)]
                acc_ref[bag, pl.ds(c * _LANES, _LANES)] = (
                    accs[c] + cv * accs[EC + c])

        def do_table(table, hi_v, par_ref, nbags, batch, nrows, aw_v, bw_v,
                     acc_ref, out_hbm):
            R = batch * nrows
            nbatches = nbags // batch

            def issue(j, buf, sem):
                pltpu.async_copy(table.at[hi_v.at[pl.ds(j * R, R)]],
                                 buf.at[pl.ds(0, R)], sem)

            def drain(buf, sem):
                pltpu.make_async_copy(table.at[pl.ds(0, R)],
                                      buf.at[pl.ds(0, R)], sem).wait()

            def compute(buf, kb):
                for jb in range(batch):
                    reduce_bag(buf, jb * nrows, par_ref, kb * batch + jb,
                               aw_v, bw_v, nrows, acc_ref)

            issue(0, rows0, sem0)
            if nbatches == 1:
                drain(rows0, sem0)
                compute(rows0, 0)
            else:
                assert nbatches % 2 == 0

                def pair(k, carry):
                    i0 = 2 * k
                    i1 = i0 + 1
                    issue(i1, rows1, sem1)
                    drain(rows0, sem0)
                    compute(rows0, i0)
                    issue(jnp.minimum(i1 + 1, nbatches - 1), rows0, sem0)
                    drain(rows1, sem1)
                    compute(rows1, i1)
                    return carry

                lax.fori_loop(0, nbatches // 2, pair, 0)
                drain(rows0, sem0)
            pltpu.sync_copy(acc_ref, out_hbm.at[wid])

        do_table(a0, qhi_v, qpar_v, QB, QBATCH, S1, qaw_v, qbw_v,
                 qacc_v, st_out)
        for table, out_hbm in ((a0, bA0), (c0, bC0), (c1, bC1), (c2, bC2)):
            do_table(table, chi_v, cpar_v, SB, SBATCH, S2, saw_v, sbw_v,
                     sacc_v, out_hbm)

    ROWS = SBATCH * S2
    f = pl.kernel(
        body,
        out_type=[jax.ShapeDtypeStruct((_NW, QB, E), jnp.float32)]
        + [jax.ShapeDtypeStruct((_NW, SB, E), jnp.float32)] * 4,
        mesh=mesh,
        scratch_types=[
            pltpu.VMEM((QB * S1,), jnp.int32),
            pltpu.VMEM((QB, S1P), jnp.float32),
            pltpu.VMEM((SB * S2,), jnp.int32),
            pltpu.VMEM((SB, S2P), jnp.float32),
            pltpu.VMEM((S1P,), jnp.float32),
            pltpu.VMEM((S1P,), jnp.float32),
            pltpu.VMEM((S2P,), jnp.float32),
            pltpu.VMEM((S2P,), jnp.float32),
            pltpu.VMEM((E,), jnp.float32),
            pltpu.VMEM((ROWS, E2), jnp.float32),
            pltpu.VMEM((ROWS, E2), jnp.float32),
            pltpu.VMEM((QB, E), jnp.float32),
            pltpu.VMEM((SB, E), jnp.float32),
            pltpu.SemaphoreType.DMA,
            pltpu.SemaphoreType.DMA,
        ],
    )
    outs = f(qhi, qpar, chi, cpar, *tabs, qaw, qbw, saw, sbw, ce)
    return (outs[0].reshape(NQB, E),) + tuple(
        o.reshape(NSB, E) for o in outs[1:])


def _hops(state0, bA0, bC0, bC1, bC2, TA, TC_, nq):
    N, E = state0.shape
    BM = bA0.shape[0]
    M = TA.shape[0]

    def body(st_ref, a_ref, c0_ref, c1_ref, c2_ref, ta_ref, tc_ref, out_ref):
        st = st_ref[...]
        ta = jnp.tile(ta_ref[...], (BM // M, 1))
        tc = jnp.tile(tc_ref[...], (BM // M, 1))
        r = lax.broadcasted_iota(jnp.int32, (N, BM), 0) // nq
        c = lax.broadcasted_iota(jnp.int32, (N, BM), 1) // M
        mask = (r == c).astype(jnp.float32)
        bags = [a_ref[...], c0_ref[...], c1_ref[...], c2_ref[...]]
        for i in range(3):
            mem = bags[i] + ta
            outp = bags[i + 1] + tc
            full = lax.dot_general(st, mem, (((1,), (1,)), ((), ())),
                                   preferred_element_type=jnp.float32)
            probs = full * mask
            resp = lax.dot_general(probs, outp, (((1,), (0,)), ((), ())),
                                   preferred_element_type=jnp.float32)
            st = st + resp
        out_ref[...] = st

    return pl.pallas_call(
        body,
        out_shape=jax.ShapeDtypeStruct((N, E), jnp.float32),
    )(state0, bA0, bC0, bC1, bC2, TA, TC_)


def _project(state, C2):
    N, E = state.shape
    Vv = C2.shape[0]
    NT = 2048
    grid = pl.cdiv(Vv, NT)

    def body(st_ref, c2t_ref, out_ref):
        out_ref[...] = lax.dot_general(
            st_ref[...], c2t_ref[...], (((1,), (0,)), ((), ())),
            preferred_element_type=jnp.float32)

    return pl.pallas_call(
        body,
        grid=(grid,),
        in_specs=[
            pl.BlockSpec((N, E), lambda i: (0, 0)),
            pl.BlockSpec((E, NT), lambda i: (0, i)),
        ],
        out_specs=pl.BlockSpec((N, NT), lambda i: (0, i)),
        out_shape=jax.ShapeDtypeStruct((N, Vv), jnp.float32),
    )(state, C2.T)


def kernel(qa_ques, ctx_ques_ans, A0, C0, C1, C2, TA, TC):
    b, nq, s = qa_ques.shape
    m, s2 = ctx_ques_ans.shape[1], ctx_ques_ans.shape[2]

    q_idx = qa_ques.reshape(b * nq, s).astype(jnp.int32)
    ctx_idx = ctx_ques_ans.reshape(b * m, s2).astype(jnp.int32)
    e = A0.shape[1]
    qaw, qbw, ce = (jnp.asarray(x) for x in _pos_weight_factors(s, e))
    saw, sbw, _ = (jnp.asarray(x) for x in _pos_weight_factors(s2, e))

    state0, bA0, bC0, bC1, bC2 = _sc_bags(
        q_idx, ctx_idx, A0, C0, C1, C2, qaw, qbw, saw, sbw, ce)
    state = _hops(state0, bA0, bC0, bC1, bC2, TA, TC, nq)
    return _project(state, C2)


# speedup vs baseline: 1.6294x; 1.1481x over previous
import functools

import numpy as np
import jax
import jax.numpy as jnp
from jax import lax
from jax.experimental import pallas as pl
from jax.experimental.pallas import tpu as pltpu
from jax.experimental.pallas import tpu_sc as plsc

_NC, _NS = 2, 16
_NW = _NC * _NS
_LANES = 16


def _pos_weights(J, d):
    j = np.arange(J, dtype=np.float32)[:, None] + 1.0
    k = np.arange(d, dtype=np.float32)[None, :] + 1.0
    return np.asarray(1.0 - j / J - (k / d) * (1.0 - 2.0 * j / J), np.float32)


def _pos_weight_factors(J, d):
    j = np.arange(J, dtype=np.float32) + 1.0
    k = np.arange(d, dtype=np.float32) + 1.0
    aw = 1.0 - j / J
    bw = 2.0 * j / J - 1.0
    ce = k / d
    return (np.asarray(aw, np.float32), np.asarray(bw, np.float32),
            np.asarray(ce, np.float32))


_PBT = 2048


def _to_pairs(t):
    Vv, E = t.shape
    BT = _PBT
    grid = -(-Vv // (2 * BT))
    HP = grid * BT
    last = -(-Vv // BT) - 1

    def body(a_ref, b_ref, o_ref):
        o_ref[...] = jnp.concatenate([a_ref[...].T, b_ref[...].T], axis=1)

    return pl.pallas_call(
        body,
        grid=(grid,),
        in_specs=[
            pl.BlockSpec((E, BT), lambda i: (0, jnp.minimum(2 * i, last))),
            pl.BlockSpec((E, BT),
                         lambda i: (0, jnp.minimum(2 * i + 1, last))),
        ],
        out_specs=pl.BlockSpec((BT, 2 * E), lambda i: (i, 0)),
        out_shape=jax.ShapeDtypeStruct((HP, 2 * E), jnp.float32),
    )(t.T, t.T)


def _sc_bags(q_idx, ctx_idx, A0, C0, C1, C2, qaw, qbw, saw, sbw, ce):
    NQB, S1 = q_idx.shape
    NSB, S2 = ctx_idx.shape
    E = A0.shape[1]
    E2 = 2 * E
    QB = NQB // _NW
    SB = NSB // _NW
    EC = E // _LANES
    QBATCH = QB
    SBATCH = 5

    S1P, S2P = S1 + _LANES, S2 + _LANES
    BT = _PBT

    def split_idx(i):
        hi = (i // (2 * BT)) * BT + (i % BT)
        par = ((i // BT) & 1).astype(jnp.float32)
        return hi, par

    qhi_f, qpar_f = split_idx(q_idx)
    chi_f, cpar_f = split_idx(ctx_idx)
    qhi = qhi_f.reshape(_NW, QB * S1)
    qpar = jnp.pad(qpar_f.reshape(_NW, QB, S1),
                   ((0, 0), (0, 0), (0, _LANES)))
    chi = chi_f.reshape(_NW, SB * S2)
    cpar = jnp.pad(cpar_f.reshape(_NW, SB, S2),
                   ((0, 0), (0, 0), (0, _LANES)))
    qaw, qbw, saw, sbw = (jnp.pad(x, (0, _LANES))
                          for x in (qaw, qbw, saw, sbw))
    tabs = [_to_pairs(t) for t in (A0, C0, C1, C2)]

    mesh = plsc.VectorSubcoreMesh(core_axis_name="c", subcore_axis_name="s")

    def launch(stabs, withq):
        NSt = len(stabs)

        def body(*refs):
            it = iter(refs)
            if withq:
                qhi_hbm, qpar_hbm = next(it), next(it)
            chi_hbm, cpar_hbm = next(it), next(it)
            s_tabs = [next(it) for _ in range(NSt)]
            if withq:
                qaw_hbm, qbw_hbm = next(it), next(it)
            saw_hbm, sbw_hbm, ce_hbm = next(it), next(it), next(it)
            if withq:
                st_out = next(it)
            s_outs = [next(it) for _ in range(NSt)]
            if withq:
                qhi_v, qpar_v, qaw_v, qbw_v, qacc_v = (
                    next(it), next(it), next(it), next(it), next(it))
            chi_v, cpar_v, saw_v, sbw_v, ce_v = (
                next(it), next(it), next(it), next(it), next(it))
            rows0, rows1, sacc_v, sem0, sem1 = (
                next(it), next(it), next(it), next(it), next(it))

            wid = lax.axis_index("s") * _NC + lax.axis_index("c")
            if withq:
                pltpu.sync_copy(qhi_hbm.at[wid], qhi_v)
                pltpu.sync_copy(qpar_hbm.at[wid], qpar_v)
                pltpu.sync_copy(qaw_hbm, qaw_v)
                pltpu.sync_copy(qbw_hbm, qbw_v)
            pltpu.sync_copy(chi_hbm.at[wid], chi_v)
            pltpu.sync_copy(cpar_hbm.at[wid], cpar_v)
            pltpu.sync_copy(saw_hbm, saw_v)
            pltpu.sync_copy(sbw_hbm, sbw_v)
            pltpu.sync_copy(ce_hbm, ce_v)

            def reduce_bag(rows_v, row0, par_ref, bag, aw_v, bw_v, nrows,
                           acc_ref):
                def sbody(s, accs):
                    p = par_ref[bag, pl.ds(s, _LANES)][0]
                    a = aw_v[pl.ds(s, _LANES)][0]
                    bwt = bw_v[pl.ds(s, _LANES)][0]
                    acc1 = list(accs[:EC])
                    acc2 = list(accs[EC:])
                    for c in range(EC):
                        h0 = rows_v[row0 + s, pl.ds(c * _LANES, _LANES)]
                        h1 = rows_v[row0 + s, pl.ds(E + c * _LANES, _LANES)]
                        h = h0 + p * (h1 - h0)
                        acc1[c] = acc1[c] + a * h
                        acc2[c] = acc2[c] + bwt * h
                    return tuple(acc1) + tuple(acc2)
                accs = lax.fori_loop(
                    0, nrows, sbody,
                    tuple(jnp.zeros((_LANES,), jnp.float32)
                          for _ in range(2 * EC)))
                for c in range(EC):
                    cv = ce_v[pl.ds(c * _LANES, _LANES)]
                    acc_ref[bag, pl.ds(c * _LANES, _LANES)] = (
                        accs[c] + cv * accs[EC + c])

            def do_table(table, hi_v, par_ref, nbags, batch, nrows, aw_v,
                         bw_v, acc_ref, out_hbm):
                R = batch * nrows
                nbatches = nbags // batch

                def issue(j, buf, sem):
                    pltpu.async_copy(table.at[hi_v.at[pl.ds(j * R, R)]],
                                     buf.at[pl.ds(0, R)], sem)

                def drain(buf, sem):
                    pltpu.make_async_copy(table.at[pl.ds(0, R)],
                                          buf.at[pl.ds(0, R)], sem).wait()

                def compute(buf, kb):
                    for jb in range(batch):
                        reduce_bag(buf, jb * nrows, par_ref,
                                   kb * batch + jb, aw_v, bw_v, nrows,
                                   acc_ref)

                issue(0, rows0, sem0)
                if nbatches == 1:
                    drain(rows0, sem0)
                    compute(rows0, 0)
                else:
                    assert nbatches % 2 == 0

                    def pair(k, carry):
                        i0 = 2 * k
                        i1 = i0 + 1
                        issue(i1, rows1, sem1)
                        drain(rows0, sem0)
                        compute(rows0, i0)
                        issue(jnp.minimum(i1 + 1, nbatches - 1), rows0,
                              sem0)
                        drain(rows1, sem1)
                        compute(rows1, i1)
                        return carry

                    lax.fori_loop(0, nbatches // 2, pair, 0)
                    drain(rows0, sem0)
                pltpu.sync_copy(acc_ref, out_hbm.at[wid])

            if withq:
                do_table(s_tabs[0], qhi_v, qpar_v, QB, QBATCH, S1, qaw_v,
                         qbw_v, qacc_v, st_out)
            for table, out_hbm in zip(s_tabs, s_outs):
                do_table(table, chi_v, cpar_v, SB, SBATCH, S2, saw_v,
                         sbw_v, sacc_v, out_hbm)

        ROWS = SBATCH * S2
        out_type = ([jax.ShapeDtypeStruct((_NW, QB, E), jnp.float32)]
                    if withq else [])
        out_type += [jax.ShapeDtypeStruct((_NW, SB, E), jnp.float32)] * NSt
        scratch = []
        if withq:
            scratch += [
                pltpu.VMEM((QB * S1,), jnp.int32),
                pltpu.VMEM((QB, S1P), jnp.float32),
                pltpu.VMEM((S1P,), jnp.float32),
                pltpu.VMEM((S1P,), jnp.float32),
                pltpu.VMEM((QB, E), jnp.float32),
            ]
        scratch += [
            pltpu.VMEM((SB * S2,), jnp.int32),
            pltpu.VMEM((SB, S2P), jnp.float32),
            pltpu.VMEM((S2P,), jnp.float32),
            pltpu.VMEM((S2P,), jnp.float32),
            pltpu.VMEM((E,), jnp.float32),
            pltpu.VMEM((ROWS, E2), jnp.float32),
            pltpu.VMEM((ROWS, E2), jnp.float32),
            pltpu.VMEM((SB, E), jnp.float32),
            pltpu.SemaphoreType.DMA,
            pltpu.SemaphoreType.DMA,
        ]
        f = pl.kernel(body, out_type=out_type, mesh=mesh,
                      scratch_types=scratch)
        args = ([qhi, qpar] if withq else []) + [chi, cpar] + stabs
        args += ([qaw, qbw] if withq else []) + [saw, sbw, ce]
        return f(*args)

    st3, bA03, bC03 = launch(tabs[:2], withq=True)
    bC13, bC23 = launch(tabs[2:], withq=False)
    return (st3.reshape(NQB, E),) + tuple(
        o.reshape(NSB, E) for o in (bA03, bC03, bC13, bC23))


def _hops(state0, bA0, bC0, bC1, bC2, TA, TC_, nq):
    N, E = state0.shape
    BM = bA0.shape[0]
    M = TA.shape[0]

    def body(st_ref, a_ref, c0_ref, c1_ref, c2_ref, ta_ref, tc_ref, out_ref):
        st = st_ref[...]
        ta = jnp.tile(ta_ref[...], (BM // M, 1))
        tc = jnp.tile(tc_ref[...], (BM // M, 1))
        r = lax.broadcasted_iota(jnp.int32, (N, BM), 0) // nq
        c = lax.broadcasted_iota(jnp.int32, (N, BM), 1) // M
        mask = (r == c).astype(jnp.float32)
        bags = [a_ref[...], c0_ref[...], c1_ref[...], c2_ref[...]]
        for i in range(3):
            mem = bags[i] + ta
            outp = bags[i + 1] + tc
            full = lax.dot_general(st, mem, (((1,), (1,)), ((), ())),
                                   preferred_element_type=jnp.float32)
            probs = full * mask
            resp = lax.dot_general(probs, outp, (((1,), (0,)), ((), ())),
                                   preferred_element_type=jnp.float32)
            st = st + resp
        out_ref[...] = st

    return pl.pallas_call(
        body,
        out_shape=jax.ShapeDtypeStruct((N, E), jnp.float32),
    )(state0, bA0, bC0, bC1, bC2, TA, TC_)


def _project(state, C2):
    N, E = state.shape
    Vv = C2.shape[0]
    NT = 2048
    grid = pl.cdiv(Vv, NT)

    def body(st_ref, c2t_ref, out_ref):
        out_ref[...] = lax.dot_general(
            st_ref[...], c2t_ref[...], (((1,), (0,)), ((), ())),
            preferred_element_type=jnp.float32)

    return pl.pallas_call(
        body,
        grid=(grid,),
        in_specs=[
            pl.BlockSpec((N, E), lambda i: (0, 0)),
            pl.BlockSpec((E, NT), lambda i: (0, i)),
        ],
        out_specs=pl.BlockSpec((N, NT), lambda i: (0, i)),
        out_shape=jax.ShapeDtypeStruct((N, Vv), jnp.float32),
    )(state, C2.T)


def kernel(qa_ques, ctx_ques_ans, A0, C0, C1, C2, TA, TC):
    b, nq, s = qa_ques.shape
    m, s2 = ctx_ques_ans.shape[1], ctx_ques_ans.shape[2]

    q_idx = qa_ques.reshape(b * nq, s).astype(jnp.int32)
    ctx_idx = ctx_ques_ans.reshape(b * m, s2).astype(jnp.int32)
    e = A0.shape[1]
    qaw, qbw, ce = (jnp.asarray(x) for x in _pos_weight_factors(s, e))
    saw, sbw, _ = (jnp.asarray(x) for x in _pos_weight_factors(s2, e))

    state0, bA0, bC0, bC1, bC2 = _sc_bags(
        q_idx, ctx_idx, A0, C0, C1, C2, qaw, qbw, saw, sbw, ce)
    state = _hops(state0, bA0, bC0, bC1, bC2, TA, TC, nq)
    return _project(state, C2)


# speedup vs baseline: 1.7976x; 1.1032x over previous
import functools

import numpy as np
import jax
import jax.numpy as jnp
from jax import lax
from jax.experimental import pallas as pl
from jax.experimental.pallas import tpu as pltpu
from jax.experimental.pallas import tpu_sc as plsc

_NC, _NS = 2, 16
_NW = _NC * _NS
_LANES = 16


def _pos_weights(J, d):
    j = np.arange(J, dtype=np.float32)[:, None] + 1.0
    k = np.arange(d, dtype=np.float32)[None, :] + 1.0
    return np.asarray(1.0 - j / J - (k / d) * (1.0 - 2.0 * j / J), np.float32)


def _pos_weight_factors(J, d):
    j = np.arange(J, dtype=np.float32) + 1.0
    k = np.arange(d, dtype=np.float32) + 1.0
    aw = 1.0 - j / J
    bw = 2.0 * j / J - 1.0
    ce = k / d
    return (np.asarray(aw, np.float32), np.asarray(bw, np.float32),
            np.asarray(ce, np.float32))


_PBT = 2048


def _to_pairs(t):
    Vv, E = t.shape
    BT = _PBT
    grid = -(-Vv // (2 * BT))
    HP = grid * BT
    last = -(-Vv // BT) - 1

    def body(a_ref, b_ref, o_ref):
        o_ref[...] = jnp.concatenate([a_ref[...].T, b_ref[...].T], axis=1)

    return pl.pallas_call(
        body,
        grid=(grid,),
        in_specs=[
            pl.BlockSpec((E, BT), lambda i: (0, jnp.minimum(2 * i, last))),
            pl.BlockSpec((E, BT),
                         lambda i: (0, jnp.minimum(2 * i + 1, last))),
        ],
        out_specs=pl.BlockSpec((BT, 2 * E), lambda i: (i, 0)),
        out_shape=jax.ShapeDtypeStruct((HP, 2 * E), jnp.float32),
    )(t.T, t.T)


def _sc_bags(q_idx, ctx_idx, A0, C0, C1, C2, qaw, qbw, saw, sbw, ce):
    NQB, S1 = q_idx.shape
    NSB, S2 = ctx_idx.shape
    E = A0.shape[1]
    E2 = 2 * E
    QB = NQB // _NW
    SB = NSB // _NW
    EC = E // _LANES
    QBATCH = QB
    SBATCH = 5

    S1P, S2P = S1 + _LANES, S2 + _LANES
    BT = _PBT

    def split_idx(i):
        hi = (i // (2 * BT)) * BT + (i % BT)
        par = ((i // BT) & 1).astype(jnp.float32)
        return hi, par

    qhi_f, qpar_f = split_idx(q_idx)
    chi_f, cpar_f = split_idx(ctx_idx)
    qhi = qhi_f.reshape(_NW, QB * S1)
    qpar = jnp.pad(qpar_f.reshape(_NW, QB, S1),
                   ((0, 0), (0, 0), (0, _LANES)))
    chi = chi_f.reshape(_NW, SB * S2)
    cpar = jnp.pad(cpar_f.reshape(_NW, SB, S2),
                   ((0, 0), (0, 0), (0, _LANES)))
    qaw, qbw, saw, sbw = (jnp.pad(x, (0, _LANES))
                          for x in (qaw, qbw, saw, sbw))
    tabs = [_to_pairs(t) for t in (A0, C0, C1, C2)]

    mesh = plsc.VectorSubcoreMesh(core_axis_name="c", subcore_axis_name="s")

    def launch(stabs, withq):
        NSt = len(stabs)

        def body(*refs):
            it = iter(refs)
            if withq:
                qhi_hbm, qpar_hbm = next(it), next(it)
            chi_hbm, cpar_hbm = next(it), next(it)
            s_tabs = [next(it) for _ in range(NSt)]
            if withq:
                qaw_hbm, qbw_hbm = next(it), next(it)
            saw_hbm, sbw_hbm, ce_hbm = next(it), next(it), next(it)
            if withq:
                st_out = next(it)
            s_outs = [next(it) for _ in range(NSt)]
            if withq:
                qhi_v, qpar_v, qaw_v, qbw_v, qacc_v = (
                    next(it), next(it), next(it), next(it), next(it))
            chi_v, cpar_v, saw_v, sbw_v, ce_v = (
                next(it), next(it), next(it), next(it), next(it))
            rows0, rows1, sacc_v, sem0, sem1 = (
                next(it), next(it), next(it), next(it), next(it))

            wid = lax.axis_index("s") * _NC + lax.axis_index("c")
            if withq:
                pltpu.sync_copy(qhi_hbm.at[wid], qhi_v)
                pltpu.sync_copy(qpar_hbm.at[wid], qpar_v)
                pltpu.sync_copy(qaw_hbm, qaw_v)
                pltpu.sync_copy(qbw_hbm, qbw_v)
            pltpu.sync_copy(chi_hbm.at[wid], chi_v)
            pltpu.sync_copy(cpar_hbm.at[wid], cpar_v)
            pltpu.sync_copy(saw_hbm, saw_v)
            pltpu.sync_copy(sbw_hbm, sbw_v)
            pltpu.sync_copy(ce_hbm, ce_v)

            def reduce_bag(rows_v, row0, par_ref, bag, aw_v, bw_v, nrows,
                           acc_ref):
                def sbody(s, accs):
                    p = par_ref[bag, pl.ds(s, _LANES)][0]
                    a = aw_v[pl.ds(s, _LANES)][0]
                    bwt = bw_v[pl.ds(s, _LANES)][0]
                    acc1 = list(accs[:EC])
                    acc2 = list(accs[EC:])
                    for c in range(EC):
                        h0 = rows_v[row0 + s, pl.ds(c * _LANES, _LANES)]
                        h1 = rows_v[row0 + s, pl.ds(E + c * _LANES, _LANES)]
                        h = h0 + p * (h1 - h0)
                        acc1[c] = acc1[c] + a * h
                        acc2[c] = acc2[c] + bwt * h
                    return tuple(acc1) + tuple(acc2)
                accs = lax.fori_loop(
                    0, nrows, sbody,
                    tuple(jnp.zeros((_LANES,), jnp.float32)
                          for _ in range(2 * EC)))
                for c in range(EC):
                    cv = ce_v[pl.ds(c * _LANES, _LANES)]
                    acc_ref[bag, pl.ds(c * _LANES, _LANES)] = (
                        accs[c] + cv * accs[EC + c])

            def do_table(table, hi_v, par_ref, nbags, batch, nrows, aw_v,
                         bw_v, acc_ref, out_hbm):
                R = batch * nrows
                nbatches = nbags // batch

                def issue(j, buf, sem):
                    pltpu.async_copy(table.at[hi_v.at[pl.ds(j * R, R)]],
                                     buf.at[pl.ds(0, R)], sem)

                def drain(buf, sem):
                    pltpu.make_async_copy(table.at[pl.ds(0, R)],
                                          buf.at[pl.ds(0, R)], sem).wait()

                def compute(buf, kb):
                    for jb in range(batch):
                        reduce_bag(buf, jb * nrows, par_ref,
                                   kb * batch + jb, aw_v, bw_v, nrows,
                                   acc_ref)

                issue(0, rows0, sem0)
                if nbatches == 1:
                    drain(rows0, sem0)
                    compute(rows0, 0)
                else:
                    assert nbatches % 2 == 0

                    def pair(k, carry):
                        i0 = 2 * k
                        i1 = i0 + 1
                        issue(i1, rows1, sem1)
                        drain(rows0, sem0)
                        compute(rows0, i0)
                        issue(jnp.minimum(i1 + 1, nbatches - 1), rows0,
                              sem0)
                        drain(rows1, sem1)
                        compute(rows1, i1)
                        return carry

                    lax.fori_loop(0, nbatches // 2, pair, 0)
                    drain(rows0, sem0)
                pltpu.sync_copy(acc_ref, out_hbm.at[wid])

            if withq:
                do_table(s_tabs[0], qhi_v, qpar_v, QB, QBATCH, S1, qaw_v,
                         qbw_v, qacc_v, st_out)
            for table, out_hbm in zip(s_tabs, s_outs):
                do_table(table, chi_v, cpar_v, SB, SBATCH, S2, saw_v,
                         sbw_v, sacc_v, out_hbm)

        ROWS = SBATCH * S2
        out_type = ([jax.ShapeDtypeStruct((_NW, QB, E), jnp.float32)]
                    if withq else [])
        out_type += [jax.ShapeDtypeStruct((_NW, SB, E), jnp.float32)] * NSt
        scratch = []
        if withq:
            scratch += [
                pltpu.VMEM((QB * S1,), jnp.int32),
                pltpu.VMEM((QB, S1P), jnp.float32),
                pltpu.VMEM((S1P,), jnp.float32),
                pltpu.VMEM((S1P,), jnp.float32),
                pltpu.VMEM((QB, E), jnp.float32),
            ]
        scratch += [
            pltpu.VMEM((SB * S2,), jnp.int32),
            pltpu.VMEM((SB, S2P), jnp.float32),
            pltpu.VMEM((S2P,), jnp.float32),
            pltpu.VMEM((S2P,), jnp.float32),
            pltpu.VMEM((E,), jnp.float32),
            pltpu.VMEM((ROWS, E2), jnp.float32),
            pltpu.VMEM((ROWS, E2), jnp.float32),
            pltpu.VMEM((SB, E), jnp.float32),
            pltpu.SemaphoreType.DMA,
            pltpu.SemaphoreType.DMA,
        ]
        f = pl.kernel(body, out_type=out_type, mesh=mesh,
                      scratch_types=scratch)
        args = ([qhi, qpar] if withq else []) + [chi, cpar] + stabs
        args += ([qaw, qbw] if withq else []) + [saw, sbw, ce]
        return f(*args)

    st3, bA03 = launch(tabs[:1], withq=True)
    (bC03,) = launch(tabs[1:2], withq=False)
    (bC13,) = launch(tabs[2:3], withq=False)
    (bC23,) = launch(tabs[3:4], withq=False)
    return (st3.reshape(NQB, E),) + tuple(
        o.reshape(NSB, E) for o in (bA03, bC03, bC13, bC23))


def _hops(state0, bA0, bC0, bC1, bC2, TA, TC_, nq):
    N, E = state0.shape
    BM = bA0.shape[0]
    M = TA.shape[0]

    def body(st_ref, a_ref, c0_ref, c1_ref, c2_ref, ta_ref, tc_ref, out_ref):
        st = st_ref[...]
        ta = jnp.tile(ta_ref[...], (BM // M, 1))
        tc = jnp.tile(tc_ref[...], (BM // M, 1))
        r = lax.broadcasted_iota(jnp.int32, (N, BM), 0) // nq
        c = lax.broadcasted_iota(jnp.int32, (N, BM), 1) // M
        mask = (r == c).astype(jnp.float32)
        bags = [a_ref[...], c0_ref[...], c1_ref[...], c2_ref[...]]
        for i in range(3):
            mem = bags[i] + ta
            outp = bags[i + 1] + tc
            full = lax.dot_general(st, mem, (((1,), (1,)), ((), ())),
                                   preferred_element_type=jnp.float32)
            probs = full * mask
            resp = lax.dot_general(probs, outp, (((1,), (0,)), ((), ())),
                                   preferred_element_type=jnp.float32)
            st = st + resp
        out_ref[...] = st

    return pl.pallas_call(
        body,
        out_shape=jax.ShapeDtypeStruct((N, E), jnp.float32),
    )(state0, bA0, bC0, bC1, bC2, TA, TC_)


def _project(state, C2):
    N, E = state.shape
    Vv = C2.shape[0]
    NT = 4096
    grid = pl.cdiv(Vv, NT)

    def body(st_ref, c2t_ref, out_ref):
        out_ref[...] = lax.dot_general(
            st_ref[...], c2t_ref[...], (((1,), (0,)), ((), ())),
            preferred_element_type=jnp.float32)

    return pl.pallas_call(
        body,
        grid=(grid,),
        in_specs=[
            pl.BlockSpec((N, E), lambda i: (0, 0)),
            pl.BlockSpec((E, NT), lambda i: (0, i)),
        ],
        out_specs=pl.BlockSpec((N, NT), lambda i: (0, i)),
        out_shape=jax.ShapeDtypeStruct((N, Vv), jnp.float32),
    )(state, C2.T)


def kernel(qa_ques, ctx_ques_ans, A0, C0, C1, C2, TA, TC):
    b, nq, s = qa_ques.shape
    m, s2 = ctx_ques_ans.shape[1], ctx_ques_ans.shape[2]

    q_idx = qa_ques.reshape(b * nq, s).astype(jnp.int32)
    ctx_idx = ctx_ques_ans.reshape(b * m, s2).astype(jnp.int32)
    e = A0.shape[1]
    qaw, qbw, ce = (jnp.asarray(x) for x in _pos_weight_factors(s, e))
    saw, sbw, _ = (jnp.asarray(x) for x in _pos_weight_factors(s2, e))

    state0, bA0, bC0, bC1, bC2 = _sc_bags(
        q_idx, ctx_idx, A0, C0, C1, C2, qaw, qbw, saw, sbw, ce)
    state = _hops(state0, bA0, bC0, bC1, bC2, TA, TC, nq)
    return _project(state, C2)


# speedup vs baseline: 1.7999x; 1.0013x over previous
import numpy as np
import jax
import jax.numpy as jnp
from jax import lax
from jax.experimental import pallas as pl
from jax.experimental.pallas import tpu as pltpu
from jax.experimental.pallas import tpu_sc as plsc

_NC, _NS = 2, 16
_NW = _NC * _NS
_LANES = 16


def _pos_weight_factors(J, d):
    j = np.arange(J, dtype=np.float32) + 1.0
    k = np.arange(d, dtype=np.float32) + 1.0
    aw = 1.0 - j / J
    bw = 2.0 * j / J - 1.0
    ce = k / d
    return (np.asarray(aw, np.float32), np.asarray(bw, np.float32),
            np.asarray(ce, np.float32))


_PBT = 2048


def _to_pairs(t):
    Vv, E = t.shape
    BT = _PBT
    grid = -(-Vv // (2 * BT))
    HP = grid * BT
    last = -(-Vv // BT) - 1

    def body(a_ref, b_ref, o_ref):
        o_ref[...] = jnp.concatenate([a_ref[...].T, b_ref[...].T], axis=1)

    return pl.pallas_call(
        body,
        grid=(grid,),
        in_specs=[
            pl.BlockSpec((E, BT), lambda i: (0, jnp.minimum(2 * i, last))),
            pl.BlockSpec((E, BT),
                         lambda i: (0, jnp.minimum(2 * i + 1, last))),
        ],
        out_specs=pl.BlockSpec((BT, 2 * E), lambda i: (i, 0)),
        out_shape=jax.ShapeDtypeStruct((HP, 2 * E), jnp.float32),
    )(t.T, t.T)


def _sc_bags(q_idx, ctx_idx, A0, C0, C1, C2, qaw, qbw, saw, sbw, ce):
    NQB, S1 = q_idx.shape
    NSB, S2 = ctx_idx.shape
    E = A0.shape[1]
    E2 = 2 * E
    QB = NQB // _NW
    SB = NSB // _NW
    EC = E // _LANES
    QBATCH = QB
    SBATCH = 5

    S1P, S2P = S1 + _LANES, S2 + _LANES
    BT = _PBT

    def split_idx(i):
        hi = (i // (2 * BT)) * BT + (i % BT)
        par = ((i // BT) & 1).astype(jnp.float32)
        return hi, par

    qhi_f, qpar_f = split_idx(q_idx)
    chi_f, cpar_f = split_idx(ctx_idx)
    qhi = qhi_f.reshape(_NW, QB * S1)
    qpar = jnp.pad(qpar_f.reshape(_NW, QB, S1),
                   ((0, 0), (0, 0), (0, _LANES)))
    chi = chi_f.reshape(_NW, SB * S2)
    cpar = jnp.pad(cpar_f.reshape(_NW, SB, S2),
                   ((0, 0), (0, 0), (0, _LANES)))
    qaw, qbw, saw, sbw = (jnp.pad(x, (0, _LANES))
                          for x in (qaw, qbw, saw, sbw))
    tabs = [_to_pairs(t) for t in (A0, C0, C1, C2)]

    mesh = plsc.VectorSubcoreMesh(core_axis_name="c", subcore_axis_name="s")

    def launch(stabs, withq):
        NSt = len(stabs)

        def body(*refs):
            it = iter(refs)
            if withq:
                qhi_hbm, qpar_hbm = next(it), next(it)
            chi_hbm, cpar_hbm = next(it), next(it)
            s_tabs = [next(it) for _ in range(NSt)]
            if withq:
                qaw_hbm, qbw_hbm = next(it), next(it)
            saw_hbm, sbw_hbm, ce_hbm = next(it), next(it), next(it)
            if withq:
                st_out = next(it)
            s_outs = [next(it) for _ in range(NSt)]
            if withq:
                qhi_v, qpar_v, qaw_v, qbw_v, qacc_v = (
                    next(it), next(it), next(it), next(it), next(it))
            chi_v, cpar_v, saw_v, sbw_v, ce_v = (
                next(it), next(it), next(it), next(it), next(it))
            rows0, rows1, sacc_v, sem0, sem1 = (
                next(it), next(it), next(it), next(it), next(it))

            wid = lax.axis_index("s") * _NC + lax.axis_index("c")
            if withq:
                pltpu.sync_copy(qhi_hbm.at[wid], qhi_v)
                pltpu.sync_copy(qpar_hbm.at[wid], qpar_v)
                pltpu.sync_copy(qaw_hbm, qaw_v)
                pltpu.sync_copy(qbw_hbm, qbw_v)
            pltpu.sync_copy(chi_hbm.at[wid], chi_v)
            pltpu.sync_copy(cpar_hbm.at[wid], cpar_v)
            pltpu.sync_copy(saw_hbm, saw_v)
            pltpu.sync_copy(sbw_hbm, sbw_v)
            pltpu.sync_copy(ce_hbm, ce_v)

            def reduce_bag(rows_v, row0, par_ref, bag, aw_v, bw_v, nrows,
                           acc_ref):
                def sbody(s, accs):
                    p = par_ref[bag, pl.ds(s, _LANES)][0]
                    a = aw_v[pl.ds(s, _LANES)][0]
                    bwt = bw_v[pl.ds(s, _LANES)][0]
                    acc1 = list(accs[:EC])
                    acc2 = list(accs[EC:])
                    for c in range(EC):
                        h0 = rows_v[row0 + s, pl.ds(c * _LANES, _LANES)]
                        h1 = rows_v[row0 + s, pl.ds(E + c * _LANES, _LANES)]
                        h = h0 + p * (h1 - h0)
                        acc1[c] = acc1[c] + a * h
                        acc2[c] = acc2[c] + bwt * h
                    return tuple(acc1) + tuple(acc2)
                accs = lax.fori_loop(
                    0, nrows, sbody,
                    tuple(jnp.zeros((_LANES,), jnp.float32)
                          for _ in range(2 * EC)))
                for c in range(EC):
                    cv = ce_v[pl.ds(c * _LANES, _LANES)]
                    acc_ref[bag, pl.ds(c * _LANES, _LANES)] = (
                        accs[c] + cv * accs[EC + c])

            def do_table(table, hi_v, par_ref, nbags, batch, nrows, aw_v,
                         bw_v, acc_ref, out_hbm):
                R = batch * nrows
                nbatches = nbags // batch

                def issue(j, buf, sem):
                    pltpu.async_copy(table.at[hi_v.at[pl.ds(j * R, R)]],
                                     buf.at[pl.ds(0, R)], sem)

                def drain(buf, sem):
                    pltpu.make_async_copy(table.at[pl.ds(0, R)],
                                          buf.at[pl.ds(0, R)], sem).wait()

                def compute(buf, kb):
                    for jb in range(batch):
                        reduce_bag(buf, jb * nrows, par_ref,
                                   kb * batch + jb, aw_v, bw_v, nrows,
                                   acc_ref)

                issue(0, rows0, sem0)
                if nbatches == 1:
                    drain(rows0, sem0)
                    compute(rows0, 0)
                else:
                    def pair(k, carry):
                        i0 = 2 * k
                        i1 = i0 + 1
                        issue(i1, rows1, sem1)
                        drain(rows0, sem0)
                        compute(rows0, i0)
                        issue(jnp.minimum(i1 + 1, nbatches - 1), rows0,
                              sem0)
                        drain(rows1, sem1)
                        compute(rows1, i1)
                        return carry

                    lax.fori_loop(0, nbatches // 2, pair, 0)
                    drain(rows0, sem0)
                    if nbatches % 2 == 1:
                        compute(rows0, nbatches - 1)
                pltpu.sync_copy(acc_ref, out_hbm.at[wid])

            if withq:
                do_table(s_tabs[0], qhi_v, qpar_v, QB, QBATCH, S1, qaw_v,
                         qbw_v, qacc_v, st_out)
            for table, out_hbm in zip(s_tabs, s_outs):
                do_table(table, chi_v, cpar_v, SB, SBATCH, S2, saw_v,
                         sbw_v, sacc_v, out_hbm)

        ROWS = SBATCH * S2
        out_type = ([jax.ShapeDtypeStruct((_NW, QB, E), jnp.float32)]
                    if withq else [])
        out_type += [jax.ShapeDtypeStruct((_NW, SB, E), jnp.float32)] * NSt
        scratch = []
        if withq:
            scratch += [
                pltpu.VMEM((QB * S1,), jnp.int32),
                pltpu.VMEM((QB, S1P), jnp.float32),
                pltpu.VMEM((S1P,), jnp.float32),
                pltpu.VMEM((S1P,), jnp.float32),
                pltpu.VMEM((QB, E), jnp.float32),
            ]
        scratch += [
            pltpu.VMEM((SB * S2,), jnp.int32),
            pltpu.VMEM((SB, S2P), jnp.float32),
            pltpu.VMEM((S2P,), jnp.float32),
            pltpu.VMEM((S2P,), jnp.float32),
            pltpu.VMEM((E,), jnp.float32),
            pltpu.VMEM((ROWS, E2), jnp.float32),
            pltpu.VMEM((ROWS, E2), jnp.float32),
            pltpu.VMEM((SB, E), jnp.float32),
            pltpu.SemaphoreType.DMA,
            pltpu.SemaphoreType.DMA,
        ]
        f = pl.kernel(body, out_type=out_type, mesh=mesh,
                      scratch_types=scratch)
        args = ([qhi, qpar] if withq else []) + [chi, cpar] + stabs
        args += ([qaw, qbw] if withq else []) + [saw, sbw, ce]
        return f(*args)

    st3, bA03 = launch(tabs[:1], withq=True)
    (bC03,) = launch(tabs[1:2], withq=False)
    (bC13,) = launch(tabs[2:3], withq=False)
    (bC23,) = launch(tabs[3:4], withq=False)
    return (st3.reshape(NQB, E),) + tuple(
        o.reshape(NSB, E) for o in (bA03, bC03, bC13, bC23))


def _hops(state0, bA0, bC0, bC1, bC2, TA, TC_, nq):
    N, E = state0.shape
    BM = bA0.shape[0]
    M = TA.shape[0]

    def body(st_ref, a_ref, c0_ref, c1_ref, c2_ref, ta_ref, tc_ref, out_ref):
        st = st_ref[...]
        ta = jnp.tile(ta_ref[...], (BM // M, 1))
        tc = jnp.tile(tc_ref[...], (BM // M, 1))
        r = lax.broadcasted_iota(jnp.int32, (N, BM), 0) // nq
        c = lax.broadcasted_iota(jnp.int32, (N, BM), 1) // M
        mask = (r == c).astype(jnp.float32)
        bags = [a_ref[...], c0_ref[...], c1_ref[...], c2_ref[...]]
        for i in range(3):
            mem = bags[i] + ta
            outp = bags[i + 1] + tc
            full = lax.dot_general(st, mem, (((1,), (1,)), ((), ())),
                                   preferred_element_type=jnp.float32)
            probs = full * mask
            resp = lax.dot_general(probs, outp, (((1,), (0,)), ((), ())),
                                   preferred_element_type=jnp.float32)
            st = st + resp
        out_ref[...] = st

    return pl.pallas_call(
        body,
        out_shape=jax.ShapeDtypeStruct((N, E), jnp.float32),
    )(state0, bA0, bC0, bC1, bC2, TA, TC_)


def _project(state, C2):
    N, E = state.shape
    Vv = C2.shape[0]
    NT = 4096
    grid = pl.cdiv(Vv, NT)

    def body(st_ref, c2t_ref, out_ref):
        out_ref[...] = lax.dot_general(
            st_ref[...], c2t_ref[...], (((1,), (0,)), ((), ())),
            preferred_element_type=jnp.float32)

    return pl.pallas_call(
        body,
        grid=(grid,),
        in_specs=[
            pl.BlockSpec((N, E), lambda i: (0, 0)),
            pl.BlockSpec((E, NT), lambda i: (0, i)),
        ],
        out_specs=pl.BlockSpec((N, NT), lambda i: (0, i)),
        out_shape=jax.ShapeDtypeStruct((N, Vv), jnp.float32),
    )(state, C2.T)


def kernel(qa_ques, ctx_ques_ans, A0, C0, C1, C2, TA, TC):
    b, nq, s = qa_ques.shape
    m, s2 = ctx_ques_ans.shape[1], ctx_ques_ans.shape[2]

    q_idx = qa_ques.reshape(b * nq, s).astype(jnp.int32)
    ctx_idx = ctx_ques_ans.reshape(b * m, s2).astype(jnp.int32)
    e = A0.shape[1]
    qaw, qbw, ce = (jnp.asarray(x) for x in _pos_weight_factors(s, e))
    saw, sbw, _ = (jnp.asarray(x) for x in _pos_weight_factors(s2, e))

    state0, bA0, bC0, bC1, bC2 = _sc_bags(
        q_idx, ctx_idx, A0, C0, C1, C2, qaw, qbw, saw, sbw, ce)
    state = _hops(state0, bA0, bC0, bC1, bC2, TA, TC, nq)
    return _project(state, C2)
